# baseline (speedup 1.0000x reference)
import functools

import jax
import jax.numpy as jnp
from jax import lax
from jax.experimental import pallas as pl
from jax.experimental.pallas import tpu as pltpu
from jax.experimental.pallas import tpu_sc as plsc

_BATCH = 16384
_HIST = 50
_F = 32

_NC = 2
_NS = 16
_NW = _NC * _NS
_EPW = _BATCH // _NW
_CB = 16
_NBUF = 4
_NCHUNK = _EPW // _CB
_NGROUP = _NCHUNK // _NBUF


def _make_gather():
    mesh = plsc.VectorSubcoreMesh(core_axis_name="c", subcore_axis_name="s")

    @functools.partial(
        pl.kernel,
        mesh=mesh,
        out_type=jax.ShapeDtypeStruct((_BATCH, _HIST, _F), jnp.float32),
        scratch_types=[
            pltpu.VMEM((_EPW, _HIST), jnp.int32),
            pltpu.VMEM((_NBUF, _CB, _HIST, _F), jnp.float32),
            pltpu.SemaphoreType.DMA((_NBUF,)),
            pltpu.SemaphoreType.DMA((_NBUF,)),
        ],
        compiler_params=pltpu.CompilerParams(use_tc_tiling_on_sc=False),
    )
    def gather_kernel(table_hbm, idx_hbm, out_hbm, idx_v, rows_v, gsem, wsem):
        wid = lax.axis_index("s") * _NC + lax.axis_index("c")
        ebase = wid * _EPW
        pltpu.sync_copy(idx_hbm.at[pl.ds(ebase, _EPW), :], idx_v)

        def group(g, carry):
            e0 = g * (_NBUF * _CB)
            for b in range(_NBUF):

                @pl.when(g > 0)
                def _drain(b=b):
                    pltpu.make_async_copy(
                        rows_v.at[b], out_hbm.at[pl.ds(ebase, _CB)], wsem.at[b]
                    ).wait()

                def fire(k, c, b=b):
                    e = e0 + b * _CB + k
                    pltpu.async_copy(
                        table_hbm.at[idx_v.at[e]], rows_v.at[b, k], gsem.at[b]
                    )
                    return c

                lax.fori_loop(0, _CB, fire, 0)
            for b in range(_NBUF):

                def drain_g(k, c, b=b):
                    e = e0 + b * _CB + k
                    pltpu.make_async_copy(
                        table_hbm.at[idx_v.at[e]], rows_v.at[b, k], gsem.at[b]
                    ).wait()
                    return c

                lax.fori_loop(0, _CB, drain_g, 0)
                pltpu.async_copy(
                    rows_v.at[b],
                    out_hbm.at[pl.ds(ebase + e0 + b * _CB, _CB)],
                    wsem.at[b],
                )
            return carry

        lax.fori_loop(0, _NGROUP, group, 0)
        for b in range(_NBUF):
            pltpu.make_async_copy(
                rows_v.at[b], out_hbm.at[pl.ds(ebase, _CB)], wsem.at[b]
            ).wait()

    return gather_kernel


_gather = _make_gather()


def kernel(inputs, embedding):
    e1, e2 = embedding[:500000], embedding[500000:]
    e1, e2 = lax.optimization_barrier((e1, e2))
    table = jnp.concatenate([e1, e2], axis=0)
    out = _gather(table, inputs.astype(jnp.int32))
    o1, o2 = out[:8192], out[8192:]
    o1, o2 = lax.optimization_barrier((o1, o2))
    return jnp.concatenate([o1, o2], axis=0)


# speedup vs baseline: 1.4055x; 1.4055x over previous
import functools

import jax
import jax.numpy as jnp
from jax import lax
from jax.experimental import pallas as pl
from jax.experimental.pallas import tpu as pltpu
from jax.experimental.pallas import tpu_sc as plsc

_BATCH = 16384
_HIST = 50
_F = 32

_NC = 2
_NS = 16
_NW = _NC * _NS
_EPW = _BATCH // _NW
_CB = 16
_NBUF = 4
_NCHUNK = _EPW // _CB
_NGROUP = _NCHUNK // _NBUF


def _make_gather():
    mesh = plsc.VectorSubcoreMesh(core_axis_name="c", subcore_axis_name="s")

    @functools.partial(
        pl.kernel,
        mesh=mesh,
        out_type=jax.ShapeDtypeStruct((_BATCH, _HIST, _F), jnp.float32),
        scratch_types=[
            pltpu.VMEM((_EPW, _HIST), jnp.int32),
            pltpu.VMEM((_NBUF, _CB, _HIST, _F), jnp.float32),
            pltpu.SemaphoreType.DMA((_NBUF,)),
            pltpu.SemaphoreType.DMA((_NBUF,)),
        ],
        compiler_params=pltpu.CompilerParams(use_tc_tiling_on_sc=False),
    )
    def gather_kernel(table_hbm, idx_hbm, out_hbm, idx_v, rows_v, gsem, wsem):
        wid = lax.axis_index("s") * _NC + lax.axis_index("c")
        ebase = wid * _EPW
        pltpu.sync_copy(idx_hbm.at[pl.ds(ebase, _EPW), :], idx_v)

        def group(g, carry):
            e0 = g * (_NBUF * _CB)
            for b in range(_NBUF):

                @pl.when(g > 0)
                def _drain(b=b):
                    pltpu.make_async_copy(
                        rows_v.at[b], out_hbm.at[pl.ds(ebase, _CB)], wsem.at[b]
                    ).wait()

                def fire(k, c, b=b):
                    e = e0 + b * _CB + k
                    pltpu.async_copy(
                        table_hbm.at[idx_v.at[e]], rows_v.at[b, k], gsem.at[b]
                    )
                    return c

                lax.fori_loop(0, _CB, fire, 0)
            for b in range(_NBUF):

                def drain_g(k, c, b=b):
                    e = e0 + b * _CB + k
                    pltpu.make_async_copy(
                        table_hbm.at[idx_v.at[e]], rows_v.at[b, k], gsem.at[b]
                    ).wait()
                    return c

                lax.fori_loop(0, _CB, drain_g, 0)
                pltpu.async_copy(
                    rows_v.at[b],
                    out_hbm.at[pl.ds(ebase + e0 + b * _CB, _CB)],
                    wsem.at[b],
                )
            return carry

        lax.fori_loop(0, _NGROUP, group, 0)
        for b in range(_NBUF):
            pltpu.make_async_copy(
                rows_v.at[b], out_hbm.at[pl.ds(ebase, _CB)], wsem.at[b]
            ).wait()

    return gather_kernel


_gather = _make_gather()


_HF = _HIST * _F
_TB = 512


def _tc_transpose_body(x_ref, o_ref):
    o_ref[...] = x_ref[...].T


_tc_transpose = pl.pallas_call(
    _tc_transpose_body,
    grid=(_BATCH // _TB,),
    in_specs=[pl.BlockSpec((_TB, _HF), lambda i: (i, 0))],
    out_specs=pl.BlockSpec((_HF, _TB), lambda i: (0, i)),
    out_shape=jax.ShapeDtypeStruct((_HF, _BATCH), jnp.float32),
)


def kernel(inputs, embedding):
    out = _gather(embedding, inputs.astype(jnp.int32))
    out2 = _tc_transpose(out.reshape(_BATCH, _HF))
    return jnp.transpose(out2.reshape(_HIST, _F, _BATCH), (2, 0, 1))


# speedup vs baseline: 1.6399x; 1.1668x over previous
import functools

import jax
import jax.numpy as jnp
from jax import lax
from jax.experimental import pallas as pl
from jax.experimental.pallas import tpu as pltpu
from jax.experimental.pallas import tpu_sc as plsc

_BATCH = 16384
_HIST = 50
_F = 32

_NC = 2
_NS = 16
_NW = _NC * _NS
_EPW = _BATCH // _NW
_CB = 16
_NBUF = 4
_NCHUNK = _EPW // _CB
_NGROUP = _NCHUNK // _NBUF


def _make_gather():
    mesh = plsc.VectorSubcoreMesh(core_axis_name="c", subcore_axis_name="s")

    @functools.partial(
        pl.kernel,
        mesh=mesh,
        out_type=jax.ShapeDtypeStruct((_BATCH, _HIST, _F), jnp.float32),
        scratch_types=[
            pltpu.VMEM((_EPW, _HIST), jnp.int32),
            pltpu.VMEM((_NBUF, _CB, _HIST, _F), jnp.float32),
            pltpu.SemaphoreType.DMA((_NBUF,)),
            pltpu.SemaphoreType.DMA((_NBUF,)),
        ],
        compiler_params=pltpu.CompilerParams(
            use_tc_tiling_on_sc=False, skip_device_barrier=True
        ),
    )
    def gather_kernel(table_hbm, idx_hbm, out_hbm, idx_v, rows_v, gsem, wsem):
        wid = lax.axis_index("s") * _NC + lax.axis_index("c")
        ebase = wid * _EPW
        pltpu.sync_copy(idx_hbm.at[pl.ds(ebase, _EPW), :], idx_v)

        def group(g, carry):
            e0 = g * (_NBUF * _CB)
            for b in range(_NBUF):

                @pl.when(g > 0)
                def _drain(b=b):
                    pltpu.make_async_copy(
                        rows_v.at[b], out_hbm.at[pl.ds(ebase, _CB)], wsem.at[b]
                    ).wait()

                def fire(k, c, b=b):
                    e = e0 + b * _CB + k
                    pltpu.async_copy(
                        table_hbm.at[idx_v.at[e]], rows_v.at[b, k], gsem.at[b]
                    )
                    return c

                lax.fori_loop(0, _CB, fire, 0)
            for b in range(_NBUF):

                def drain_g(k, c, b=b):
                    e = e0 + b * _CB + k
                    pltpu.make_async_copy(
                        table_hbm.at[idx_v.at[e]], rows_v.at[b, k], gsem.at[b]
                    ).wait()
                    return c

                lax.fori_loop(0, _CB, drain_g, 0)
                pltpu.async_copy(
                    rows_v.at[b],
                    out_hbm.at[pl.ds(ebase + e0 + b * _CB, _CB)],
                    wsem.at[b],
                )
            return carry

        lax.fori_loop(0, _NGROUP, group, 0)
        for b in range(_NBUF):
            pltpu.make_async_copy(
                rows_v.at[b], out_hbm.at[pl.ds(ebase, _CB)], wsem.at[b]
            ).wait()

    return gather_kernel


_gather = _make_gather()


def kernel(inputs, embedding):
    return _gather(embedding, inputs.astype(jnp.int32))


# speedup vs baseline: 1.6411x; 1.0007x over previous
import functools

import jax
import jax.numpy as jnp
from jax import lax
from jax.experimental import pallas as pl
from jax.experimental.pallas import tpu as pltpu
from jax.experimental.pallas import tpu_sc as plsc

_BATCH = 16384
_HIST = 50
_F = 32

_NC = 2
_NS = 16
_NW = _NC * _NS
_EPW = _BATCH // _NW
_CB = 16
_NBUF = 4
_NCHUNK = _EPW // _CB
_NGROUP = _NCHUNK // _NBUF


def _make_gather():
    mesh = plsc.VectorSubcoreMesh(core_axis_name="c", subcore_axis_name="s")

    @functools.partial(
        pl.kernel,
        mesh=mesh,
        out_type=jax.ShapeDtypeStruct((_BATCH, _HIST, _F), jnp.float32),
        scratch_types=[
            pltpu.VMEM((_EPW, _HIST), jnp.int32),
            pltpu.VMEM((_NBUF, _CB, _HIST, _F), jnp.float32),
            pltpu.SemaphoreType.DMA((_NBUF,)),
            pltpu.SemaphoreType.DMA((_NBUF,)),
        ],
        compiler_params=pltpu.CompilerParams(use_tc_tiling_on_sc=False),
    )
    def gather_kernel(table_hbm, idx_hbm, out_hbm, idx_v, rows_v, gsem, wsem):
        wid = lax.axis_index("s") * _NC + lax.axis_index("c")
        ebase = wid * _EPW
        pltpu.sync_copy(idx_hbm.at[pl.ds(ebase, _EPW), :], idx_v)

        def group(g, carry):
            e0 = g * (_NBUF * _CB)
            for b in range(_NBUF):

                @pl.when(g > 0)
                def _drain(b=b):
                    pltpu.make_async_copy(
                        rows_v.at[b], out_hbm.at[pl.ds(ebase, _CB)], wsem.at[b]
                    ).wait()

                def fire(k, c, b=b):
                    e = e0 + b * _CB + k
                    pltpu.async_copy(
                        table_hbm.at[idx_v.at[e]], rows_v.at[b, k], gsem.at[b]
                    )
                    return c

                lax.fori_loop(0, _CB, fire, 0)
            for b in range(_NBUF):

                def drain_g(k, c, b=b):
                    e = e0 + b * _CB + k
                    pltpu.make_async_copy(
                        table_hbm.at[idx_v.at[e]], rows_v.at[b, k], gsem.at[b]
                    ).wait()
                    return c

                lax.fori_loop(0, _CB, drain_g, 0)
                pltpu.async_copy(
                    rows_v.at[b],
                    out_hbm.at[pl.ds(ebase + e0 + b * _CB, _CB)],
                    wsem.at[b],
                )
            return carry

        lax.fori_loop(0, _NGROUP, group, 0)
        for b in range(_NBUF):
            pltpu.make_async_copy(
                rows_v.at[b], out_hbm.at[pl.ds(ebase, _CB)], wsem.at[b]
            ).wait()

    return gather_kernel


_gather = _make_gather()


def kernel(inputs, embedding):
    return _gather(embedding, inputs.astype(jnp.int32))
